# baseline (speedup 1.0000x reference)
import jax
import jax.numpy as jnp
from jax import lax
from jax.experimental import pallas as pl
from jax.experimental.pallas import tpu as pltpu
from jax.experimental.pallas import tpu_sc as plsc

_N = 100000
_NB = 256
_BS = 1.0 / _NB
_INV_BS = float(_NB)
_MIN_SIZE = _BS * 1.4142135
_SCALE = 1.0 / (_BS * _BS * 100.0)
_NW = 32
_CHUNK = 3128
_LAST = _N - (_NW - 1) * _CHUNK
_GROUPS = _CHUNK // 16
_TAIL = _CHUNK - _GROUPS * 16
_NBINS = _NB * _NB
_KMAX = 7


def _sc_body(x_hbm, y_hbm, sx_hbm, sy_hbm, w_hbm, out_hbm,
             xv, yv, sxv, syv, wv, acc):
    wid = lax.axis_index("s") * 2 + lax.axis_index("c")
    is_last = wid == _NW - 1
    base = wid * _CHUNK

    zeros16 = jnp.zeros((16,), jnp.float32)
    for buf in (xv, yv, sxv, syv, wv):
        buf[pl.ds(3120, 16)] = zeros16

    @pl.when(is_last)
    def _():
        for buf in (xv, yv, sxv, syv, wv):
            for r in range(3024, 3136, 16):
                buf[pl.ds(r, 16)] = zeros16

    @pl.when(jnp.logical_not(is_last))
    def _():
        pltpu.sync_copy(x_hbm.at[pl.ds(base, _CHUNK)], xv.at[pl.ds(0, _CHUNK)])
        pltpu.sync_copy(y_hbm.at[pl.ds(base, _CHUNK)], yv.at[pl.ds(0, _CHUNK)])
        pltpu.sync_copy(sx_hbm.at[pl.ds(base, _CHUNK)], sxv.at[pl.ds(0, _CHUNK)])
        pltpu.sync_copy(sy_hbm.at[pl.ds(base, _CHUNK)], syv.at[pl.ds(0, _CHUNK)])
        pltpu.sync_copy(w_hbm.at[pl.ds(base, _CHUNK)], wv.at[pl.ds(0, _CHUNK)])

    @pl.when(is_last)
    def _():
        pltpu.sync_copy(x_hbm.at[pl.ds(base, _LAST)], xv.at[pl.ds(0, _LAST)])
        pltpu.sync_copy(y_hbm.at[pl.ds(base, _LAST)], yv.at[pl.ds(0, _LAST)])
        pltpu.sync_copy(sx_hbm.at[pl.ds(base, _LAST)], sxv.at[pl.ds(0, _LAST)])
        pltpu.sync_copy(sy_hbm.at[pl.ds(base, _LAST)], syv.at[pl.ds(0, _LAST)])
        pltpu.sync_copy(w_hbm.at[pl.ds(base, _LAST)], wv.at[pl.ds(0, _LAST)])

    zero16 = jnp.zeros((16,), jnp.float32)

    @plsc.parallel_loop(0, 512, unroll=4)
    def zero_row(i):
        for k in range(8):
            acc[i, pl.ds(k * 16, 16)] = zero16

    @plsc.parallel_loop(0, _GROUPS + 1, unroll=2)
    def group_body(g):
        s = g * 16
        x = xv[pl.ds(s, 16)]
        y = yv[pl.ds(s, 16)]
        sx = jnp.maximum(sxv[pl.ds(s, 16)], _MIN_SIZE)
        sy = jnp.maximum(syv[pl.ds(s, 16)], _MIN_SIZE)
        w = wv[pl.ds(s, 16)]
        hx = 0.5 * sx
        hy = 0.5 * sy
        x_min = x - hx
        x_max = x + hx
        y_min = y - hy
        y_max = y + hy
        dens = (w * _SCALE) / (sx * sy)
        ix0 = (x_min * _INV_BS + 1024.0).astype(jnp.int32) - 1024
        iy0 = (y_min * _INV_BS + 1024.0).astype(jnp.int32) - 1024
        xlo0 = ix0.astype(jnp.float32) * _BS
        ylo0 = iy0.astype(jnp.float32) * _BS

        row2 = []
        px = []
        mx = []
        for dx in range(_KMAX):
            bx = ix0 + dx
            lo = xlo0 + dx * _BS
            ox = jnp.maximum(
                jnp.minimum(x_max, lo + _BS) - jnp.maximum(x_min, lo), 0.0)
            px.append(dens * ox)
            mx.append(lax.bitcast_convert_type(bx, jnp.uint32) < _NB)
            row2.append(bx * 2)

        hi = []
        lo_col = []
        py = []
        my = []
        for dy in range(_KMAX):
            by = iy0 + dy
            lo = ylo0 + dy * _BS
            oy = jnp.maximum(
                jnp.minimum(y_max, lo + _BS) - jnp.maximum(y_min, lo), 0.0)
            py.append(oy)
            my.append(lax.bitcast_convert_type(by, jnp.uint32) < _NB)
            hi.append(by >> 7)
            lo_col.append(by & 127)

        for dx in range(_KMAX):
            for dy in range(_KMAX):
                row = row2[dx] + hi[dy]
                val = px[dx] * py[dy]
                m = mx[dx] & my[dy]
                plsc.addupdate_scatter(acc, [row, lo_col[dy]], val, mask=m)

    pltpu.sync_copy(acc, out_hbm.at[wid])


@jax.jit
def _sc_maps(x, y, sx, sy, w):
    mesh = plsc.VectorSubcoreMesh(core_axis_name="c", subcore_axis_name="s")
    return pl.kernel(
        _sc_body,
        out_type=jax.ShapeDtypeStruct((_NW, 2 * _NB, _NB // 2), jnp.float32),
        mesh=mesh,
        compiler_params=pltpu.CompilerParams(needs_layout_passes=False),
        scratch_types=[
            pltpu.VMEM((_GROUPS * 16 + 16,), jnp.float32),
            pltpu.VMEM((_GROUPS * 16 + 16,), jnp.float32),
            pltpu.VMEM((_GROUPS * 16 + 16,), jnp.float32),
            pltpu.VMEM((_GROUPS * 16 + 16,), jnp.float32),
            pltpu.VMEM((_GROUPS * 16 + 16,), jnp.float32),
            pltpu.VMEM((2 * _NB, _NB // 2), jnp.float32),
        ],
    )(x, y, sx, sy, w)


def _reduce_body(maps_ref, out_ref):
    out_ref[...] = jnp.sum(maps_ref[...], axis=0).reshape(_NB, _NB)


@jax.jit
def _reduce(maps):
    return pl.pallas_call(
        _reduce_body,
        out_shape=jax.ShapeDtypeStruct((_NB, _NB), jnp.float32),
    )(maps)


def kernel(inst_sizes, inst_pos, inst_pin_weights):
    maps = _sc_maps(inst_pos[:, 0], inst_pos[:, 1],
                    inst_sizes[:, 0], inst_sizes[:, 1], inst_pin_weights)
    return _reduce(maps)


# speedup vs baseline: 1.4325x; 1.4325x over previous
import jax
import jax.numpy as jnp
from jax import lax
from jax.experimental import pallas as pl
from jax.experimental.pallas import tpu as pltpu
from jax.experimental.pallas import tpu_sc as plsc

_N = 100000
_NB = 256
_BS = 1.0 / _NB
_INV_BS = float(_NB)
_MIN_SIZE = _BS * 1.4142135
_SCALE = 1.0 / (_BS * _BS * 100.0)
_NW = 32
_CHUNK = 3128
_LAST = _N - (_NW - 1) * _CHUNK
_GROUPS = _CHUNK // 16
_TAIL = _CHUNK - _GROUPS * 16
_NBINS = _NB * _NB
_KMAX = 7


def _sc_body(x_hbm, y_hbm, sx_hbm, sy_hbm, w_hbm, out_hbm,
             xv, yv, sxv, syv, wv, acc):
    wid = lax.axis_index("s") * 2 + lax.axis_index("c")
    is_last = wid == _NW - 1
    base = wid * _CHUNK

    zeros16 = jnp.zeros((16,), jnp.float32)
    for buf in (xv, yv, sxv, syv, wv):
        buf[pl.ds(3120, 16)] = zeros16

    @pl.when(is_last)
    def _():
        for buf in (xv, yv, sxv, syv, wv):
            for r in range(3024, 3136, 16):
                buf[pl.ds(r, 16)] = zeros16

    @pl.when(jnp.logical_not(is_last))
    def _():
        pltpu.sync_copy(x_hbm.at[pl.ds(base, _CHUNK)], xv.at[pl.ds(0, _CHUNK)])
        pltpu.sync_copy(y_hbm.at[pl.ds(base, _CHUNK)], yv.at[pl.ds(0, _CHUNK)])
        pltpu.sync_copy(sx_hbm.at[pl.ds(base, _CHUNK)], sxv.at[pl.ds(0, _CHUNK)])
        pltpu.sync_copy(sy_hbm.at[pl.ds(base, _CHUNK)], syv.at[pl.ds(0, _CHUNK)])
        pltpu.sync_copy(w_hbm.at[pl.ds(base, _CHUNK)], wv.at[pl.ds(0, _CHUNK)])

    @pl.when(is_last)
    def _():
        pltpu.sync_copy(x_hbm.at[pl.ds(base, _LAST)], xv.at[pl.ds(0, _LAST)])
        pltpu.sync_copy(y_hbm.at[pl.ds(base, _LAST)], yv.at[pl.ds(0, _LAST)])
        pltpu.sync_copy(sx_hbm.at[pl.ds(base, _LAST)], sxv.at[pl.ds(0, _LAST)])
        pltpu.sync_copy(sy_hbm.at[pl.ds(base, _LAST)], syv.at[pl.ds(0, _LAST)])
        pltpu.sync_copy(w_hbm.at[pl.ds(base, _LAST)], wv.at[pl.ds(0, _LAST)])

    zero16 = jnp.zeros((16,), jnp.float32)

    @plsc.parallel_loop(0, 512, unroll=4)
    def zero_row(i):
        for k in range(8):
            acc[i, pl.ds(k * 16, 16)] = zero16

    @plsc.parallel_loop(0, _GROUPS + 1)
    def group_body(g):
        s = g * 16
        x = xv[pl.ds(s, 16)]
        y = yv[pl.ds(s, 16)]
        sx = jnp.maximum(sxv[pl.ds(s, 16)], _MIN_SIZE)
        sy = jnp.maximum(syv[pl.ds(s, 16)], _MIN_SIZE)
        w = wv[pl.ds(s, 16)]
        hx = 0.5 * sx
        hy = 0.5 * sy
        x_min = x - hx
        x_max = x + hx
        y_min = y - hy
        y_max = y + hy
        dens = (w * _SCALE) / (sx * sy)
        ix0 = (x_min * _INV_BS + 1024.0).astype(jnp.int32) - 1024
        iy0 = (y_min * _INV_BS + 1024.0).astype(jnp.int32) - 1024
        xlo0 = ix0.astype(jnp.float32) * _BS
        ylo0 = iy0.astype(jnp.float32) * _BS

        row2 = []
        px = []
        mx = []
        for dx in range(_KMAX):
            bx = ix0 + dx
            lo = xlo0 + dx * _BS
            ox = jnp.maximum(
                jnp.minimum(x_max, lo + _BS) - jnp.maximum(x_min, lo), 0.0)
            px.append(dens * ox)
            mx.append(lax.bitcast_convert_type(bx, jnp.uint32) < _NB)
            row2.append(bx * 2)

        hi = []
        lo_col = []
        py = []
        my = []
        for dy in range(_KMAX):
            by = iy0 + dy
            lo = ylo0 + dy * _BS
            oy = jnp.maximum(
                jnp.minimum(y_max, lo + _BS) - jnp.maximum(y_min, lo), 0.0)
            py.append(oy)
            my.append(lax.bitcast_convert_type(by, jnp.uint32) < _NB)
            hi.append(by >> 7)
            lo_col.append(by & 127)

        for dx in range(_KMAX):
            for dy in range(_KMAX):
                row = row2[dx] + hi[dy]
                val = px[dx] * py[dy]
                m = mx[dx] & my[dy]
                plsc.addupdate_scatter(acc, [row, lo_col[dy]], val, mask=m)

    pltpu.sync_copy(acc, out_hbm.at[wid])


@jax.jit
def _sc_maps(x, y, sx, sy, w):
    mesh = plsc.VectorSubcoreMesh(core_axis_name="c", subcore_axis_name="s")
    return pl.kernel(
        _sc_body,
        out_type=jax.ShapeDtypeStruct((_NW, 2 * _NB, _NB // 2), jnp.float32),
        mesh=mesh,
        compiler_params=pltpu.CompilerParams(needs_layout_passes=False),
        scratch_types=[
            pltpu.VMEM((_GROUPS * 16 + 16,), jnp.float32),
            pltpu.VMEM((_GROUPS * 16 + 16,), jnp.float32),
            pltpu.VMEM((_GROUPS * 16 + 16,), jnp.float32),
            pltpu.VMEM((_GROUPS * 16 + 16,), jnp.float32),
            pltpu.VMEM((_GROUPS * 16 + 16,), jnp.float32),
            pltpu.VMEM((2 * _NB, _NB // 2), jnp.float32),
        ],
    )(x, y, sx, sy, w)


def _reduce_body(maps_ref, out_ref):
    out_ref[...] = jnp.sum(maps_ref[...], axis=0).reshape(_NB, _NB)


@jax.jit
def _reduce(maps):
    return pl.pallas_call(
        _reduce_body,
        out_shape=jax.ShapeDtypeStruct((_NB, _NB), jnp.float32),
    )(maps)


def kernel(inst_sizes, inst_pos, inst_pin_weights):
    maps = _sc_maps(inst_pos[:, 0], inst_pos[:, 1],
                    inst_sizes[:, 0], inst_sizes[:, 1], inst_pin_weights)
    return _reduce(maps)


# speedup vs baseline: 1.4497x; 1.0120x over previous
import jax
import jax.numpy as jnp
from jax import lax
from jax.experimental import pallas as pl
from jax.experimental.pallas import tpu as pltpu
from jax.experimental.pallas import tpu_sc as plsc

_N = 100000
_NB = 256
_BS = 1.0 / _NB
_INV_BS = float(_NB)
_MIN_SIZE = _BS * 1.4142135
_SCALE = 1.0 / (_BS * _BS * 100.0)
_NW = 32
_CHUNK = 3128
_LAST = _N - (_NW - 1) * _CHUNK
_GROUPS = _CHUNK // 16
_TAIL = _CHUNK - _GROUPS * 16
_NBINS = _NB * _NB
_KMAX = 7


def _sc_body(x_hbm, y_hbm, sx_hbm, sy_hbm, w_hbm, out_hbm,
             xv, yv, sxv, syv, wv, acc):
    wid = lax.axis_index("s") * 2 + lax.axis_index("c")
    is_last = wid == _NW - 1
    base = wid * _CHUNK

    zeros16 = jnp.zeros((16,), jnp.float32)
    for buf in (xv, yv, sxv, syv, wv):
        buf[pl.ds(3120, 16)] = zeros16

    @pl.when(is_last)
    def _():
        for buf in (xv, yv, sxv, syv, wv):
            for r in range(3024, 3136, 16):
                buf[pl.ds(r, 16)] = zeros16

    @pl.when(jnp.logical_not(is_last))
    def _():
        pltpu.sync_copy(x_hbm.at[pl.ds(base, _CHUNK)], xv.at[pl.ds(0, _CHUNK)])
        pltpu.sync_copy(y_hbm.at[pl.ds(base, _CHUNK)], yv.at[pl.ds(0, _CHUNK)])
        pltpu.sync_copy(sx_hbm.at[pl.ds(base, _CHUNK)], sxv.at[pl.ds(0, _CHUNK)])
        pltpu.sync_copy(sy_hbm.at[pl.ds(base, _CHUNK)], syv.at[pl.ds(0, _CHUNK)])
        pltpu.sync_copy(w_hbm.at[pl.ds(base, _CHUNK)], wv.at[pl.ds(0, _CHUNK)])

    @pl.when(is_last)
    def _():
        pltpu.sync_copy(x_hbm.at[pl.ds(base, _LAST)], xv.at[pl.ds(0, _LAST)])
        pltpu.sync_copy(y_hbm.at[pl.ds(base, _LAST)], yv.at[pl.ds(0, _LAST)])
        pltpu.sync_copy(sx_hbm.at[pl.ds(base, _LAST)], sxv.at[pl.ds(0, _LAST)])
        pltpu.sync_copy(sy_hbm.at[pl.ds(base, _LAST)], syv.at[pl.ds(0, _LAST)])
        pltpu.sync_copy(w_hbm.at[pl.ds(base, _LAST)], wv.at[pl.ds(0, _LAST)])

    zero16 = jnp.zeros((16,), jnp.float32)

    @plsc.parallel_loop(0, 512, unroll=4)
    def zero_row(i):
        for k in range(8):
            acc[i, pl.ds(k * 16, 16)] = zero16

    @plsc.parallel_loop(0, _GROUPS + 1)
    def group_body(g):
        s = g * 16
        x = xv[pl.ds(s, 16)]
        y = yv[pl.ds(s, 16)]
        sx = jnp.maximum(sxv[pl.ds(s, 16)], _MIN_SIZE)
        sy = jnp.maximum(syv[pl.ds(s, 16)], _MIN_SIZE)
        w = wv[pl.ds(s, 16)]
        hx = 0.5 * sx
        hy = 0.5 * sy
        x_min = x - hx
        x_max = x + hx
        y_min = y - hy
        y_max = y + hy
        dens = (w * _SCALE) / (sx * sy)
        ix0 = (x_min * _INV_BS + 1024.0).astype(jnp.int32) - 1024
        iy0 = (y_min * _INV_BS + 1024.0).astype(jnp.int32) - 1024
        xlo0 = ix0.astype(jnp.float32) * _BS
        ylo0 = iy0.astype(jnp.float32) * _BS

        hi = []
        lo_col = []
        py = []
        my = []
        for dy in range(_KMAX):
            by = iy0 + dy
            lo = ylo0 + dy * _BS
            oy = jnp.maximum(
                jnp.minimum(y_max, lo + _BS) - jnp.maximum(y_min, lo), 0.0)
            py.append(oy)
            my.append(lax.bitcast_convert_type(by, jnp.uint32) < _NB)
            hi.append(by >> 7)
            lo_col.append(by & 127)

        for dx in range(_KMAX):
            bx = ix0 + dx
            lo = xlo0 + dx * _BS
            ox = jnp.maximum(
                jnp.minimum(x_max, lo + _BS) - jnp.maximum(x_min, lo), 0.0)
            pxd = dens * ox
            mxd = lax.bitcast_convert_type(bx, jnp.uint32) < _NB
            row2 = bx * 2
            for dy in range(_KMAX):
                row = row2 + hi[dy]
                val = pxd * py[dy]
                m = mxd & my[dy]
                plsc.addupdate_scatter(acc, [row, lo_col[dy]], val, mask=m)

    pltpu.sync_copy(acc, out_hbm.at[wid])


@jax.jit
def _sc_maps(x, y, sx, sy, w):
    mesh = plsc.VectorSubcoreMesh(core_axis_name="c", subcore_axis_name="s")
    return pl.kernel(
        _sc_body,
        out_type=jax.ShapeDtypeStruct((_NW, 2 * _NB, _NB // 2), jnp.float32),
        mesh=mesh,
        compiler_params=pltpu.CompilerParams(needs_layout_passes=False),
        scratch_types=[
            pltpu.VMEM((_GROUPS * 16 + 16,), jnp.float32),
            pltpu.VMEM((_GROUPS * 16 + 16,), jnp.float32),
            pltpu.VMEM((_GROUPS * 16 + 16,), jnp.float32),
            pltpu.VMEM((_GROUPS * 16 + 16,), jnp.float32),
            pltpu.VMEM((_GROUPS * 16 + 16,), jnp.float32),
            pltpu.VMEM((2 * _NB, _NB // 2), jnp.float32),
        ],
    )(x, y, sx, sy, w)


def _reduce_body(maps_ref, out_ref):
    out_ref[...] = jnp.sum(maps_ref[...], axis=0).reshape(_NB, _NB)


@jax.jit
def _reduce(maps):
    return pl.pallas_call(
        _reduce_body,
        out_shape=jax.ShapeDtypeStruct((_NB, _NB), jnp.float32),
    )(maps)


def kernel(inst_sizes, inst_pos, inst_pin_weights):
    maps = _sc_maps(inst_pos[:, 0], inst_pos[:, 1],
                    inst_sizes[:, 0], inst_sizes[:, 1], inst_pin_weights)
    return _reduce(maps)
